# baseline (speedup 1.0000x reference)
import jax
import jax.numpy as jnp
from jax import lax
from jax.experimental import pallas as pl
from jax.experimental.pallas import tpu as pltpu
from jax.experimental.pallas import tpu_sc as plsc

DIM = 32
L = 50
NC = 2
NS = 16
NW = NC * NS
BC = 16
CL = BC * L

_GATHER_DNUMS = lax.GatherDimensionNumbers(
    offset_dims=(), collapsed_slice_dims=(0,), start_index_map=(0,))


def _perm(vec, idx):
    return lax.gather(vec, idx.reshape(16, 1), _GATHER_DNUMS, (1,),
                      mode=lax.GatherScatterMode.PROMISE_IN_BOUNDS)


def _bcast_lane(vec, lane_idx):
    return _perm(vec, jnp.full((16,), lane_idx, jnp.int32))


def _allmax(v):
    lane = lax.broadcasted_iota(jnp.int32, (16,), 0)
    for k in (1, 2, 4, 8):
        v = jnp.maximum(v, _perm(v, lane ^ k))
    return v


def _allsum(v):
    lane = lax.broadcasted_iota(jnp.int32, (16,), 0)
    for k in (1, 2, 4, 8):
        v = v + _perm(v, lane ^ k)
    return v


_SUBS = []
_off = 0
while _off < CL:
    _n = min(128, CL - _off)
    _SUBS.append((_off, _n))
    _off += _n


def _net_body(dr_hbm, de_hbm, rel_hbm, pid_hbm, nid_hbm, ent_hbm, edge_hbm,
              relt_hbm, out_hbm, pos_hbm, neg_hbm,
              dr_v, de_v, rel_i, pid_v, nid_v,
              w_v, e_v, r_v, p_v, n_v, out_v, sem):
    B = rel_hbm.shape[0]
    rows_per_w = B // NW
    nchunk = rows_per_w // BC
    wid = lax.axis_index("s") * NC + lax.axis_index("c")
    wstart = wid * rows_per_w

    lane = lax.broadcasted_iota(jnp.int32, (16,), 0)
    neg_inf = jnp.float32(-jnp.inf)

    def chunk_body(ci, _):
        base = wstart + ci * BC
        fbase = base * L
        pltpu.sync_copy(dr_hbm.at[pl.ds(fbase, CL)], dr_v)
        pltpu.sync_copy(de_hbm.at[pl.ds(fbase, CL)], de_v)
        pltpu.sync_copy(rel_hbm.at[pl.ds(base, BC)], rel_i)
        pltpu.sync_copy(pid_hbm.at[pl.ds(base, BC)], pid_v)
        pltpu.sync_copy(nid_hbm.at[pl.ds(base, BC)], nid_v)
        descs = []
        for off, n in _SUBS:
            descs.append(pltpu.async_copy(
                edge_hbm.at[dr_v.at[pl.ds(off, n)]],
                w_v.at[pl.ds(off, n)], sem))
            descs.append(pltpu.async_copy(
                ent_hbm.at[de_v.at[pl.ds(off, n)]],
                e_v.at[pl.ds(off, n), :], sem))
        descs.append(pltpu.async_copy(relt_hbm.at[rel_i], r_v, sem))
        descs.append(pltpu.async_copy(ent_hbm.at[pid_v], p_v, sem))
        descs.append(pltpu.async_copy(ent_hbm.at[nid_v], n_v, sem))
        for d in descs:
            d.wait()

        def row_body(b, _):
            off = b * L
            c0 = w_v[pl.ds(off, 16)]
            c1 = w_v[pl.ds(off + 16, 16)]
            c2 = w_v[pl.ds(off + 32, 16)]
            c3 = w_v[pl.ds(off + 48, 16)]
            c3 = jnp.where(lane < (L - 48), c3, neg_inf)
            m = _allmax(jnp.maximum(jnp.maximum(c0, c1), jnp.maximum(c2, c3)))
            x0 = jnp.exp(c0 - m)
            x1 = jnp.exp(c1 - m)
            x2 = jnp.exp(c2 - m)
            x3 = jnp.exp(c3 - m)
            s = _allsum(x0 + x1 + x2 + x3)
            inv = jnp.float32(1.0) / s
            wch = (x0 * inv, x1 * inv, x2 * inv, x3 * inv)
            acc0 = r_v[b, pl.ds(0, 16)]
            acc1 = r_v[b, pl.ds(16, 16)]
            for l in range(L):
                wl = _bcast_lane(wch[l // 16], l % 16)
                acc0 = acc0 + wl * e_v[off + l, pl.ds(0, 16)]
                acc1 = acc1 + wl * e_v[off + l, pl.ds(16, 16)]
            out_v[b, pl.ds(0, 16)] = acc0
            out_v[b, pl.ds(16, 16)] = acc1
            return 0

        lax.fori_loop(0, BC, row_body, 0)

        pltpu.sync_copy(out_v, out_hbm.at[pl.ds(base, BC), :])
        pltpu.sync_copy(p_v, pos_hbm.at[pl.ds(base, BC), :])
        pltpu.sync_copy(n_v, neg_hbm.at[pl.ds(base, BC), :])
        return 0

    lax.fori_loop(0, nchunk, chunk_body, 0)


def kernel(data_r, data_e, rel, pos_id, neg_id, entity_table, edge_table,
           rel_table):
    B = data_e.shape[0]
    dr_flat = data_r.astype(jnp.int32).reshape(-1)
    de_flat = data_e.astype(jnp.int32).reshape(-1)
    rel = rel.astype(jnp.int32)
    pos_id = pos_id.astype(jnp.int32)
    neg_id = neg_id.astype(jnp.int32)
    edge1d = edge_table.reshape(-1)

    mesh = plsc.VectorSubcoreMesh(core_axis_name="c", subcore_axis_name="s")
    f32 = jnp.float32
    run = pl.kernel(
        _net_body,
        out_type=(
            jax.ShapeDtypeStruct((B, DIM), f32),
            jax.ShapeDtypeStruct((B, DIM), f32),
            jax.ShapeDtypeStruct((B, DIM), f32),
        ),
        mesh=mesh,
        scratch_types=[
            pltpu.VMEM((CL,), jnp.int32),
            pltpu.VMEM((CL,), jnp.int32),
            pltpu.VMEM((BC,), jnp.int32),
            pltpu.VMEM((BC,), jnp.int32),
            pltpu.VMEM((BC,), jnp.int32),
            pltpu.VMEM((CL + 16,), f32),
            pltpu.VMEM((CL, DIM), f32),
            pltpu.VMEM((BC, DIM), f32),
            pltpu.VMEM((BC, DIM), f32),
            pltpu.VMEM((BC, DIM), f32),
            pltpu.VMEM((BC, DIM), f32),
            pltpu.SemaphoreType.DMA,
        ],
        compiler_params=pltpu.CompilerParams(use_tc_tiling_on_sc=False),
    )
    out_t, pos_out, neg_out = run(dr_flat, de_flat, rel, pos_id, neg_id,
                                  entity_table, edge1d, rel_table)
    return (out_t, pos_out, neg_out)
